# Initial kernel scaffold; baseline (speedup 1.0000x reference)
import jax
import jax.numpy as jnp
from jax.experimental import pallas as pl


def kernel(inputs, weight):
    raise NotImplementedError("write your pallas kernel here")


# speedup vs baseline: 1.0800x; 1.0800x over previous
import jax
import jax.numpy as jnp
from jax.experimental import pallas as pl
from jax.experimental.pallas import tpu as pltpu

_B = 16
_D = 64
_N = 1024
_K = 1024
_EPS = 1e-12


def _vq_kernel(x_ref, w_ref, q_ref, loss_ref, perp_ref, acc_ref, cnt_ref):
    b = pl.program_id(0)

    x = x_ref[0]
    w = w_ref[...]

    wn = w / jnp.maximum(jnp.sqrt(jnp.sum(w * w, axis=1, keepdims=True)), _EPS)
    xn = x / jnp.maximum(jnp.sqrt(jnp.sum(x * x, axis=0, keepdims=True)), _EPS)

    scores = jax.lax.dot_general(
        xn, wn, (((0,), (1,)), ((), ())),
        preferred_element_type=jnp.float32)
    d = 1.0 - scores

    dmin = jnp.min(d, axis=1, keepdims=True)
    iota_k = jax.lax.broadcasted_iota(jnp.int32, (_N, _K), 1)
    idx = jnp.min(jnp.where(d == dmin, iota_k, _K), axis=1, keepdims=True)
    onehot = (iota_k == idx).astype(jnp.float32)

    q = jax.lax.dot_general(
        w, onehot, (((0,), (1,)), ((), ())),
        preferred_element_type=jnp.float32)
    q_ref[0] = x + (q - x)

    sq = jnp.sum((q - x) ** 2)
    counts = jnp.sum(onehot, axis=0).reshape(1, _K)

    @pl.when(b == 0)
    def _init():
        acc_ref[0, 0] = 0.0
        cnt_ref[...] = jnp.zeros((1, _K), jnp.float32)

    acc_ref[0, 0] += sq
    cnt_ref[...] += counts

    @pl.when(b == _B - 1)
    def _finish():
        total = jnp.float32(_B * _D * _N)
        loss_ref[...] = (1.25 * acc_ref[0, 0] / total).reshape(1, 1)
        probs = cnt_ref[...] / jnp.float32(_B * _N)
        ent = -jnp.sum(probs * jnp.log(probs + 1e-10))
        perp_ref[...] = jnp.exp(ent).reshape(1, 1)


def kernel(inputs, weight):
    B, D, H, W = inputs.shape
    x = inputs.reshape(B, D, H * W)

    q, loss, perp = pl.pallas_call(
        _vq_kernel,
        grid=(B,),
        in_specs=[
            pl.BlockSpec((1, D, H * W), lambda b: (b, 0, 0)),
            pl.BlockSpec((_K, D), lambda b: (0, 0)),
        ],
        out_specs=[
            pl.BlockSpec((1, D, H * W), lambda b: (b, 0, 0)),
            pl.BlockSpec((1, 1), lambda b: (0, 0)),
            pl.BlockSpec((1, 1), lambda b: (0, 0)),
        ],
        out_shape=[
            jax.ShapeDtypeStruct((B, D, H * W), jnp.float32),
            jax.ShapeDtypeStruct((1, 1), jnp.float32),
            jax.ShapeDtypeStruct((1, 1), jnp.float32),
        ],
        scratch_shapes=[
            pltpu.SMEM((1, 1), jnp.float32),
            pltpu.VMEM((1, _K), jnp.float32),
        ],
    )(x, weight)

    return (q.reshape(B, D, H, W), loss[0, 0], perp[0, 0])


# speedup vs baseline: 1.0826x; 1.0025x over previous
import jax
import jax.numpy as jnp
from jax.experimental import pallas as pl
from jax.experimental.pallas import tpu as pltpu

_B = 16
_D = 64
_N = 1024
_K = 1024
_EPS = 1e-12


def _vq_kernel(x_ref, w_ref, q_ref, loss_ref, perp_ref, acc_ref, cnt_ref):
    b = pl.program_id(0)

    x = x_ref[0]
    w = w_ref[...]

    wn = w / jnp.maximum(jnp.sqrt(jnp.sum(w * w, axis=1, keepdims=True)), _EPS)
    xn = x / jnp.maximum(jnp.sqrt(jnp.sum(x * x, axis=0, keepdims=True)), _EPS)

    scores = jax.lax.dot_general(
        xn, wn, (((0,), (1,)), ((), ())),
        preferred_element_type=jnp.float32)
    d = 1.0 - scores

    idx = jnp.argmin(d, axis=1)
    iota_k = jax.lax.broadcasted_iota(jnp.int32, (_N, _K), 1)
    onehot = (iota_k == idx[:, None]).astype(jnp.float32)

    q = jax.lax.dot_general(
        w, onehot, (((0,), (1,)), ((), ())),
        preferred_element_type=jnp.float32)
    q_ref[0] = x + (q - x)

    sq = jnp.sum((q - x) ** 2)
    counts = jax.lax.dot_general(
        jnp.ones((1, _N), jnp.float32), onehot, (((1,), (0,)), ((), ())),
        preferred_element_type=jnp.float32)

    @pl.when(b == 0)
    def _init():
        acc_ref[0, 0] = 0.0
        cnt_ref[...] = jnp.zeros((1, _K), jnp.float32)

    acc_ref[0, 0] += sq
    cnt_ref[...] += counts

    @pl.when(b == _B - 1)
    def _finish():
        total = jnp.float32(_B * _D * _N)
        loss_ref[...] = (1.25 * acc_ref[0, 0] / total).reshape(1, 1)
        probs = cnt_ref[...] / jnp.float32(_B * _N)
        ent = -jnp.sum(probs * jnp.log(probs + 1e-10))
        perp_ref[...] = jnp.exp(ent).reshape(1, 1)


def kernel(inputs, weight):
    B, D, H, W = inputs.shape
    x = inputs.reshape(B, D, H * W)

    q, loss, perp = pl.pallas_call(
        _vq_kernel,
        grid=(B,),
        in_specs=[
            pl.BlockSpec((1, D, H * W), lambda b: (b, 0, 0)),
            pl.BlockSpec((_K, D), lambda b: (0, 0)),
        ],
        out_specs=[
            pl.BlockSpec((1, D, H * W), lambda b: (b, 0, 0)),
            pl.BlockSpec((1, 1), lambda b: (0, 0)),
            pl.BlockSpec((1, 1), lambda b: (0, 0)),
        ],
        out_shape=[
            jax.ShapeDtypeStruct((B, D, H * W), jnp.float32),
            jax.ShapeDtypeStruct((1, 1), jnp.float32),
            jax.ShapeDtypeStruct((1, 1), jnp.float32),
        ],
        scratch_shapes=[
            pltpu.SMEM((1, 1), jnp.float32),
            pltpu.VMEM((1, _K), jnp.float32),
        ],
    )(x, weight)

    return (q.reshape(B, D, H, W), loss[0, 0], perp[0, 0])
